# baseline (speedup 1.0000x reference)
import jax
import jax.numpy as jnp
from jax.experimental import pallas as pl
from jax.experimental.pallas import tpu as pltpu


def _mlp_kernel(x_ref, w1p_ref, b1p_ref, w2t_ref, b2t_ref, o_ref):
    tb = x_ref.shape[0]
    tb2 = tb // 2
    kin = x_ref.shape[1]
    pad = 128 - kin

    xa = x_ref[0:tb2, :]
    xb = x_ref[tb2:, :]
    x2 = jnp.concatenate(
        [
            jnp.pad(xa, ((0, 0), (0, pad))),
            jnp.pad(xb, ((0, 0), (0, pad))),
        ],
        axis=1,
    )
    h2 = jnp.dot(x2, w1p_ref[...], preferred_element_type=jnp.float32)
    h2 = jnp.maximum(h2 + b1p_ref[...], 0.0)

    w2t = w2t_ref[0:5, :]
    dims = (((1,), (1,)), ((), ()))
    yta = jax.lax.dot_general(
        w2t, h2[:, 0:64], dims, preferred_element_type=jnp.float32
    )
    ytb = jax.lax.dot_general(
        w2t, h2[:, 64:128], dims, preferred_element_type=jnp.float32
    )
    yt = jnp.concatenate([yta, ytb], axis=1) + b2t_ref[0:5, 0:1]
    o_ref[...] = yt


def kernel(x, w1, b1, w2, b2, *, block_batch=4096):
    B, K = x.shape
    H = w1.shape[1]
    O = w2.shape[1]

    w1p = (
        jnp.zeros((256, 256), jnp.float32)
        .at[0:K, 0:H]
        .set(w1)
        .at[128 : 128 + K, H : 2 * H]
        .set(w1)
    )
    b1p = jnp.zeros((1, 256), jnp.float32).at[:, 0:H].set(b1).at[:, H : 2 * H].set(b1)
    w2t = jnp.zeros((8, H), jnp.float32).at[0:O, :].set(w2.T)
    b2t = jnp.zeros((8, 128), jnp.float32).at[0:O, 0:1].set(b2.T)

    TB = min(block_batch, B)
    n = pl.cdiv(B, TB)

    cost = pl.CostEstimate(
        flops=2 * B * (K * H + H * O),
        transcendentals=0,
        bytes_accessed=4 * (B * (K + O) + 256 * 256 + H * O),
    )

    yt = pl.pallas_call(
        _mlp_kernel,
        out_shape=jax.ShapeDtypeStruct((O, B), jnp.float32),
        grid=(n,),
        in_specs=[
            pl.BlockSpec((TB, K), lambda i: (i, 0)),
            pl.BlockSpec((256, 256), lambda i: (0, 0)),
            pl.BlockSpec((1, 256), lambda i: (0, 0)),
            pl.BlockSpec((8, 64), lambda i: (0, 0)),
            pl.BlockSpec((8, 128), lambda i: (0, 0)),
        ],
        out_specs=pl.BlockSpec((O, TB), lambda i: (0, i)),
        compiler_params=pltpu.CompilerParams(
            dimension_semantics=("parallel",)
        ),
        cost_estimate=cost,
    )(x, w1p, b1p, w2t, b2t)

    return yt.T


# speedup vs baseline: 1.1344x; 1.1344x over previous
import jax
import jax.numpy as jnp
from jax.experimental import pallas as pl
from jax.experimental.pallas import tpu as pltpu


def _mlp_kernel(x_ref, w1p_ref, b1p_ref, w2t_ref, b2t_ref, o_ref):
    tb = x_ref.shape[0]
    tb2 = tb // 2
    kin = x_ref.shape[1]
    pad = 128 - kin

    xa = x_ref[0:tb2, :]
    xb = x_ref[tb2:, :]
    x2 = jnp.concatenate(
        [
            jnp.pad(xa, ((0, 0), (0, pad))),
            jnp.pad(xb, ((0, 0), (0, pad))),
        ],
        axis=1,
    )
    h2 = jnp.dot(x2, w1p_ref[...], preferred_element_type=jnp.float32)
    h2 = jnp.maximum(h2 + b1p_ref[...], 0.0)

    w2t = w2t_ref[0:5, :]
    dims = (((1,), (1,)), ((), ()))
    yta = jax.lax.dot_general(
        w2t, h2[:, 0:64], dims, preferred_element_type=jnp.float32
    )
    ytb = jax.lax.dot_general(
        w2t, h2[:, 64:128], dims, preferred_element_type=jnp.float32
    )
    yt = jnp.concatenate([yta, ytb], axis=1) + b2t_ref[0:5, 0:1]
    o_ref[...] = yt


def kernel(x, w1, b1, w2, b2, *, block_batch=16384):
    B, K = x.shape
    H = w1.shape[1]
    O = w2.shape[1]

    w1p = (
        jnp.zeros((256, 256), jnp.float32)
        .at[0:K, 0:H]
        .set(w1)
        .at[128 : 128 + K, H : 2 * H]
        .set(w1)
    )
    b1p = jnp.zeros((1, 256), jnp.float32).at[:, 0:H].set(b1).at[:, H : 2 * H].set(b1)
    w2t = jnp.zeros((8, H), jnp.float32).at[0:O, :].set(w2.T)
    b2t = jnp.zeros((8, 128), jnp.float32).at[0:O, 0:1].set(b2.T)

    TB = min(block_batch, B)
    n = pl.cdiv(B, TB)

    cost = pl.CostEstimate(
        flops=2 * B * (K * H + H * O),
        transcendentals=0,
        bytes_accessed=4 * (B * (K + O) + 256 * 256 + H * O),
    )

    yt = pl.pallas_call(
        _mlp_kernel,
        out_shape=jax.ShapeDtypeStruct((O, B), jnp.float32),
        grid=(n,),
        in_specs=[
            pl.BlockSpec((TB, K), lambda i: (i, 0)),
            pl.BlockSpec((256, 256), lambda i: (0, 0)),
            pl.BlockSpec((1, 256), lambda i: (0, 0)),
            pl.BlockSpec((8, 64), lambda i: (0, 0)),
            pl.BlockSpec((8, 128), lambda i: (0, 0)),
        ],
        out_specs=pl.BlockSpec((O, TB), lambda i: (0, i)),
        compiler_params=pltpu.CompilerParams(
            dimension_semantics=("parallel",)
        ),
        cost_estimate=cost,
    )(x, w1p, b1p, w2t, b2t)

    return yt.T
